# baseline (speedup 1.0000x reference)
import dataclasses
import functools

import jax
import jax.numpy as jnp
from jax import lax
from jax.experimental import pallas as pl
from jax.experimental.pallas import tpu as pltpu
from jax.experimental.pallas import tpu_sc as plsc

NC, NS, L = 2, 16, 16
NW = NC * NS
B = 16384
D = 64
BPW = B // NW


def _compiler_params():
    cp = pltpu.CompilerParams()
    fields = pltpu.CompilerParams.__dataclass_fields__
    if "needs_layout_passes" in fields:
        cp = dataclasses.replace(cp, needs_layout_passes=False)
    if "use_tc_tiling_on_sc" in fields:
        cp = dataclasses.replace(cp, use_tc_tiling_on_sc=False)
    return cp


def kernel(user, item, user_table, item_table):
    mesh = plsc.VectorSubcoreMesh(core_axis_name="c", subcore_axis_name="s")

    @functools.partial(
        pl.kernel,
        mesh=mesh,
        compiler_params=_compiler_params(),
        out_type=jax.ShapeDtypeStruct((B,), jnp.float32),
        scratch_types=[
            pltpu.VMEM((BPW,), jnp.int32),
            pltpu.VMEM((BPW,), jnp.int32),
            pltpu.VMEM((BPW, D), jnp.float32),
            pltpu.VMEM((BPW, D), jnp.float32),
            pltpu.VMEM((BPW,), jnp.float32),
            pltpu.SemaphoreType.DMA,
        ],
    )
    def k(user_hbm, item_hbm, ut_hbm, it_hbm, out_hbm,
          uidx_v, iidx_v, urows_v, irows_v, out_v, sem):
        wid = lax.axis_index("s") * NC + lax.axis_index("c")
        base = wid * BPW
        pltpu.sync_copy(user_hbm.at[pl.ds(base, BPW)], uidx_v)
        pltpu.sync_copy(item_hbm.at[pl.ds(base, BPW)], iidx_v)
        cu = pltpu.async_copy(ut_hbm.at[uidx_v], urows_v, sem)
        ci = pltpu.async_copy(it_hbm.at[iidx_v], irows_v, sem)
        cu.wait()
        ci.wait()

        lanes = lax.iota(jnp.int32, L)

        @pl.loop(0, BPW, step=L)
        def _(b0):
            out_vec = jnp.zeros((L,), jnp.float32)
            for j in range(L):
                b = b0 + j
                acc = urows_v[b, pl.ds(0, L)] * irows_v[b, pl.ds(0, L)]
                for k0 in range(L, D, L):
                    acc = acc + urows_v[b, pl.ds(k0, L)] * irows_v[b, pl.ds(k0, L)]
                out_vec = jnp.where(lanes == j, jnp.sum(acc), out_vec)
            out_v[pl.ds(b0, L)] = out_vec

        pltpu.sync_copy(out_v, out_hbm.at[pl.ds(base, BPW)])

    return k(user, item, user_table, item_table)
